# baseline (speedup 1.0000x reference)
import jax
import jax.numpy as jnp
from jax import lax
from jax.experimental import pallas as pl
from jax.experimental.pallas import tpu as pltpu
from jax.experimental.pallas import tpu_sc as plsc

N = 10000
E = 160000
D = 256
H = 128
NC = 2
NS = 16
CH = 128
NP = 10240
RPT = NP // NS
EPAD = 163840
EPT = EPAD // NS
NFULL = EPT // CH
EPT2 = EPAD // (NC * NS)
NFULL2 = EPT2 // CH


def _make_sc_agg(compute_deg):
  mesh = plsc.VectorSubcoreMesh(
      core_axis_name="c", subcore_axis_name="s", num_cores=NC, num_subcores=NS)

  if compute_deg:
    out_type = (jax.ShapeDtypeStruct((NC, NP, H), jnp.float32),
                jax.ShapeDtypeStruct((NC, NP, H), jnp.float32))
  else:
    out_type = jax.ShapeDtypeStruct((NC, NP, H), jnp.float32)

  scratch = [
      pltpu.VMEM((NFULL2, CH), jnp.int32),
      pltpu.VMEM((NFULL2, CH), jnp.int32),
      pltpu.VMEM((CH, H), jnp.float32),
      pltpu.VMEM((CH, H), jnp.float32),
      pltpu.VMEM((CH,), jnp.int32),
      pltpu.VMEM((CH,), jnp.int32),
      pltpu.VMEM((CH,), jnp.int32),
      pltpu.VMEM((CH,), jnp.int32),
      pltpu.VMEM_SHARED((NP, H), jnp.float32),
      pltpu.SemaphoreType.DMA,
      pltpu.SemaphoreType.DMA,
  ]

  def body(h2, src2, dst2, z128, *rest):
    if compute_deg:
      (ones_hbm, agg, deg,
       srcs_v, dsts_v, rows0_v, rows1_v, srcb0, srcb1, dstb0, dstb1,
       acc_sh, sem0, sem1) = rest
    else:
      (agg, srcs_v, dsts_v, rows0_v, rows1_v, srcb0, srcb1, dstb0, dstb1,
       acc_sh, sem0, sem1) = rest

    def vfill(buf, tbl, row):
      for j in range(CH // 16):
        buf[pl.ds(16 * j, 16)] = tbl[row, pl.ds(16 * j, 16)]

    cid = lax.axis_index("c")
    sid = lax.axis_index("s")
    r0 = sid * RPT
    rsl = pl.ds(r0, RPT)

    pltpu.sync_copy(z128.at[rsl], acc_sh.at[rsl])
    plsc.subcore_barrier()

    tab = h2.at[cid]

    def pair(g, carry):
      k0 = 2 * g
      vfill(srcb0, srcs_v, k0)
      a0 = pltpu.async_copy(tab.at[srcb0], rows0_v, sem0)
      vfill(srcb1, srcs_v, k0 + 1)
      a1 = pltpu.async_copy(tab.at[srcb1], rows1_v, sem1)
      vfill(dstb0, dsts_v, k0)
      a0.wait()
      pltpu.sync_copy(rows0_v, acc_sh.at[dstb0], add=True)
      vfill(dstb1, dsts_v, k0 + 1)
      a1.wait()
      pltpu.sync_copy(rows1_v, acc_sh.at[dstb1], add=True)
      return carry

    for half in range(2):
      pltpu.sync_copy(src2.at[pl.ds(sid * NFULL + half * NFULL2, NFULL2)],
                      srcs_v)
      pltpu.sync_copy(dst2.at[pl.ds(sid * NFULL + half * NFULL2, NFULL2)],
                      dsts_v)
      lax.fori_loop(0, NFULL2 // 2, pair, 0)

    plsc.subcore_barrier()
    pltpu.sync_copy(acc_sh.at[rsl], agg.at[cid].at[rsl])

    if compute_deg:
      plsc.subcore_barrier()
      pltpu.sync_copy(z128.at[rsl], acc_sh.at[rsl])
      plsc.subcore_barrier()

      pltpu.sync_copy(ones_hbm, rows0_v)
      pltpu.sync_copy(dst2.at[pl.ds(cid * (NS * NFULL2) + sid * NFULL2,
                                    NFULL2)], dsts_v)

      def dpair(g, carry):
        k0 = 2 * g
        vfill(dstb0, dsts_v, k0)
        a0 = pltpu.make_async_copy(rows0_v, acc_sh.at[dstb0], sem0)
        a0.start(add=True)
        vfill(dstb1, dsts_v, k0 + 1)
        a1 = pltpu.make_async_copy(rows0_v, acc_sh.at[dstb1], sem1)
        a1.start(add=True)
        a0.wait()
        a1.wait()
        return carry

      lax.fori_loop(0, NFULL2 // 2, dpair, 0)

      plsc.subcore_barrier()
      pltpu.sync_copy(acc_sh.at[rsl], deg.at[cid].at[rsl])

  return pl.kernel(body, out_type=out_type, mesh=mesh,
                   scratch_types=tuple(scratch))


_sc_agg_deg = _make_sc_agg(True)
_sc_agg = _make_sc_agg(False)


BN = 1000


def _tc1_body(agg0, agg1, deg0, deg1, x, wl, b, wr, h0, h1):
  d = jnp.maximum(deg0[:, 0:1] + deg1[:, 0:1], 1.0)
  mean = jnp.concatenate([agg0[...], agg1[...]], axis=-1) / d
  out = (jnp.dot(mean, wl[...], preferred_element_type=jnp.float32)
         + jnp.dot(x[...], wr[...], preferred_element_type=jnp.float32)
         + b[...])
  nrm = jnp.sqrt(jnp.sum(out * out, axis=-1, keepdims=True))
  out = out / jnp.maximum(nrm, 1e-12)
  h = jnp.maximum(out, 0.0)
  h0[...] = h[:, :H]
  h1[...] = h[:, H:]


def _tc2_body(agg0, agg1, deg0, deg1, h0, h1, wl, b, wr, out):
  d = jnp.maximum(deg0[:, 0:1] + deg1[:, 0:1], 1.0)
  mean = jnp.concatenate([agg0[...], agg1[...]], axis=-1) / d
  hh = jnp.concatenate([h0[...], h1[...]], axis=-1)
  out[...] = (jnp.dot(mean, wl[...], preferred_element_type=jnp.float32)
              + jnp.dot(hh, wr[...], preferred_element_type=jnp.float32)
              + b[...])


def _row_spec(w):
  return pl.BlockSpec((BN, w), lambda i: (i, 0))


def _full_spec(r, c):
  return pl.BlockSpec((r, c), lambda i: (0, 0))


_tc1 = pl.pallas_call(
    _tc1_body,
    grid=(N // BN,),
    in_specs=[_row_spec(H), _row_spec(H), _row_spec(H), _row_spec(H),
              _row_spec(D), _full_spec(D, D), _full_spec(1, D),
              _full_spec(D, D)],
    out_specs=[_row_spec(H), _row_spec(H)],
    out_shape=[jax.ShapeDtypeStruct((N, H), jnp.float32),
               jax.ShapeDtypeStruct((N, H), jnp.float32)],
)

_tc2 = pl.pallas_call(
    _tc2_body,
    grid=(N // BN,),
    in_specs=[_row_spec(H), _row_spec(H), _row_spec(H), _row_spec(H),
              _row_spec(H), _row_spec(H), _full_spec(D, D), _full_spec(1, D),
              _full_spec(D, D)],
    out_specs=_row_spec(D),
    out_shape=jax.ShapeDtypeStruct((N, D), jnp.float32),
)


def _pad_edges(src, dst):
  npad = EPAD - E
  psrc = jnp.zeros((npad,), jnp.int32)
  pdst = (N + jnp.arange(npad, dtype=jnp.int32) % (NP - N)).astype(jnp.int32)
  return (jnp.concatenate([src, psrc]).reshape(EPAD // CH, CH),
          jnp.concatenate([dst, pdst]).reshape(EPAD // CH, CH))


def kernel(x, edge_index, W1l, b1, W1r, W3l, b3, W3r):
  src, dst = _pad_edges(edge_index[0], edge_index[1])
  x2 = jnp.stack([x[:, :H], x[:, H:]])
  z128 = jnp.zeros((NP, H), jnp.float32)
  ones = jnp.ones((CH, H), jnp.float32)

  agg, deg = _sc_agg_deg(x2, src, dst, z128, ones)
  h0, h1 = _tc1(agg[0], agg[1], deg[0], deg[1], x,
                W1l.T, b1[None, :], W1r.T)
  h2 = jnp.stack([h0, h1])
  aggb = _sc_agg(h2, src, dst, z128)
  out = _tc2(aggb[0], aggb[1], deg[0], deg[1], h0, h1,
             W3l.T, b3[None, :], W3r.T)
  return out


# speedup vs baseline: 1.0554x; 1.0554x over previous
import jax
import jax.numpy as jnp
from jax import lax
from jax.experimental import pallas as pl
from jax.experimental.pallas import tpu as pltpu
from jax.experimental.pallas import tpu_sc as plsc

N = 10000
E = 160000
D = 256
H = 128
NC = 2
NS = 16
CH = 128
NP = 10240
RPT = NP // NS
EPAD = 163840
EPT = EPAD // NS
NFULL = EPT // CH
EPT2 = EPAD // (NC * NS)
NFULL2 = EPT2 // CH


def _make_sc_agg(compute_deg):
  mesh = plsc.VectorSubcoreMesh(
      core_axis_name="c", subcore_axis_name="s", num_cores=NC, num_subcores=NS)

  if compute_deg:
    out_type = (jax.ShapeDtypeStruct((NC, NP, H), jnp.float32),
                jax.ShapeDtypeStruct((NC, NP, H), jnp.float32))
  else:
    out_type = jax.ShapeDtypeStruct((NC, NP, H), jnp.float32)

  scratch = [
      pltpu.VMEM((NFULL2, CH), jnp.int32),
      pltpu.VMEM((NFULL2, CH), jnp.int32),
      pltpu.VMEM((CH, H), jnp.float32),
      pltpu.VMEM((CH, H), jnp.float32),
      pltpu.VMEM((CH,), jnp.int32),
      pltpu.VMEM((CH,), jnp.int32),
      pltpu.VMEM((CH,), jnp.int32),
      pltpu.VMEM((CH,), jnp.int32),
      pltpu.VMEM_SHARED((NP, H), jnp.float32),
      pltpu.SemaphoreType.DMA,
      pltpu.SemaphoreType.DMA,
      pltpu.SemaphoreType.DMA,
      pltpu.SemaphoreType.DMA,
  ]

  def body(h2, src2, dst2, z128, *rest):
    if compute_deg:
      (ones_hbm, agg, deg,
       srcs_v, dsts_v, rows0_v, rows1_v, srcb0, srcb1, dstb0, dstb1,
       acc_sh, sem0, sem1, sem2, sem3) = rest
    else:
      (agg, srcs_v, dsts_v, rows0_v, rows1_v, srcb0, srcb1, dstb0, dstb1,
       acc_sh, sem0, sem1, sem2, sem3) = rest

    def vfill(buf, tbl, row):
      for j in range(CH // 16):
        buf[pl.ds(16 * j, 16)] = tbl[row, pl.ds(16 * j, 16)]

    cid = lax.axis_index("c")
    sid = lax.axis_index("s")
    r0 = sid * RPT
    rsl = pl.ds(r0, RPT)

    pltpu.sync_copy(z128.at[rsl], acc_sh.at[rsl])
    plsc.subcore_barrier()

    tab = h2.at[cid]

    def pair(g, carry):
      k0 = 2 * g

      @pl.when(g > 0)
      def _():
        pltpu.make_async_copy(rows0_v, acc_sh.at[dstb0], sem2).wait()

      vfill(srcb0, srcs_v, k0)
      a0 = pltpu.async_copy(tab.at[srcb0], rows0_v, sem0)

      @pl.when(g > 0)
      def _():
        pltpu.make_async_copy(rows1_v, acc_sh.at[dstb1], sem3).wait()

      vfill(srcb1, srcs_v, k0 + 1)
      a1 = pltpu.async_copy(tab.at[srcb1], rows1_v, sem1)
      a0.wait()
      vfill(dstb0, dsts_v, k0)
      s0 = pltpu.make_async_copy(rows0_v, acc_sh.at[dstb0], sem2)
      s0.start(add=True)
      a1.wait()
      vfill(dstb1, dsts_v, k0 + 1)
      s1 = pltpu.make_async_copy(rows1_v, acc_sh.at[dstb1], sem3)
      s1.start(add=True)
      return carry

    for half in range(2):
      pltpu.sync_copy(src2.at[pl.ds(sid * NFULL + half * NFULL2, NFULL2)],
                      srcs_v)
      pltpu.sync_copy(dst2.at[pl.ds(sid * NFULL + half * NFULL2, NFULL2)],
                      dsts_v)
      lax.fori_loop(0, NFULL2 // 2, pair, 0)
      pltpu.make_async_copy(rows0_v, acc_sh.at[dstb0], sem2).wait()
      pltpu.make_async_copy(rows1_v, acc_sh.at[dstb1], sem3).wait()

    plsc.subcore_barrier()
    pltpu.sync_copy(acc_sh.at[rsl], agg.at[cid].at[rsl])

    if compute_deg:
      plsc.subcore_barrier()
      pltpu.sync_copy(z128.at[rsl], acc_sh.at[rsl])
      plsc.subcore_barrier()

      pltpu.sync_copy(ones_hbm, rows0_v)
      pltpu.sync_copy(dst2.at[pl.ds(cid * (NS * NFULL2) + sid * NFULL2,
                                    NFULL2)], dsts_v)

      def dpair(g, carry):
        k0 = 2 * g
        vfill(dstb0, dsts_v, k0)
        a0 = pltpu.make_async_copy(rows0_v, acc_sh.at[dstb0], sem0)
        a0.start(add=True)
        vfill(dstb1, dsts_v, k0 + 1)
        a1 = pltpu.make_async_copy(rows0_v, acc_sh.at[dstb1], sem1)
        a1.start(add=True)
        a0.wait()
        a1.wait()
        return carry

      lax.fori_loop(0, NFULL2 // 2, dpair, 0)

      plsc.subcore_barrier()
      pltpu.sync_copy(acc_sh.at[rsl], deg.at[cid].at[rsl])

  return pl.kernel(body, out_type=out_type, mesh=mesh,
                   scratch_types=tuple(scratch))


_sc_agg_deg = _make_sc_agg(True)
_sc_agg = _make_sc_agg(False)


BN = 1000


def _tc1_body(agg0, agg1, deg0, deg1, x, wl, b, wr, h0, h1):
  d = jnp.maximum(deg0[:, 0:1] + deg1[:, 0:1], 1.0)
  mean = jnp.concatenate([agg0[...], agg1[...]], axis=-1) / d
  out = (jnp.dot(mean, wl[...], preferred_element_type=jnp.float32)
         + jnp.dot(x[...], wr[...], preferred_element_type=jnp.float32)
         + b[...])
  nrm = jnp.sqrt(jnp.sum(out * out, axis=-1, keepdims=True))
  out = out / jnp.maximum(nrm, 1e-12)
  h = jnp.maximum(out, 0.0)
  h0[...] = h[:, :H]
  h1[...] = h[:, H:]


def _tc2_body(agg0, agg1, deg0, deg1, h0, h1, wl, b, wr, out):
  d = jnp.maximum(deg0[:, 0:1] + deg1[:, 0:1], 1.0)
  mean = jnp.concatenate([agg0[...], agg1[...]], axis=-1) / d
  hh = jnp.concatenate([h0[...], h1[...]], axis=-1)
  out[...] = (jnp.dot(mean, wl[...], preferred_element_type=jnp.float32)
              + jnp.dot(hh, wr[...], preferred_element_type=jnp.float32)
              + b[...])


def _row_spec(w):
  return pl.BlockSpec((BN, w), lambda i: (i, 0))


def _full_spec(r, c):
  return pl.BlockSpec((r, c), lambda i: (0, 0))


_tc1 = pl.pallas_call(
    _tc1_body,
    grid=(N // BN,),
    in_specs=[_row_spec(H), _row_spec(H), _row_spec(H), _row_spec(H),
              _row_spec(D), _full_spec(D, D), _full_spec(1, D),
              _full_spec(D, D)],
    out_specs=[_row_spec(H), _row_spec(H)],
    out_shape=[jax.ShapeDtypeStruct((N, H), jnp.float32),
               jax.ShapeDtypeStruct((N, H), jnp.float32)],
)

_tc2 = pl.pallas_call(
    _tc2_body,
    grid=(N // BN,),
    in_specs=[_row_spec(H), _row_spec(H), _row_spec(H), _row_spec(H),
              _row_spec(H), _row_spec(H), _full_spec(D, D), _full_spec(1, D),
              _full_spec(D, D)],
    out_specs=_row_spec(D),
    out_shape=jax.ShapeDtypeStruct((N, D), jnp.float32),
)


def _pad_edges(src, dst):
  npad = EPAD - E
  psrc = jnp.zeros((npad,), jnp.int32)
  pdst = (N + jnp.arange(npad, dtype=jnp.int32) % (NP - N)).astype(jnp.int32)
  return (jnp.concatenate([src, psrc]).reshape(EPAD // CH, CH),
          jnp.concatenate([dst, pdst]).reshape(EPAD // CH, CH))


def kernel(x, edge_index, W1l, b1, W1r, W3l, b3, W3r):
  src, dst = _pad_edges(edge_index[0], edge_index[1])
  x2 = jnp.stack([x[:, :H], x[:, H:]])
  z128 = jnp.zeros((NP, H), jnp.float32)
  ones = jnp.ones((CH, H), jnp.float32)

  agg, deg = _sc_agg_deg(x2, src, dst, z128, ones)
  h0, h1 = _tc1(agg[0], agg[1], deg[0], deg[1], x,
                W1l.T, b1[None, :], W1r.T)
  h2 = jnp.stack([h0, h1])
  aggb = _sc_agg(h2, src, dst, z128)
  out = _tc2(aggb[0], aggb[1], deg[0], deg[1], h0, h1,
             W3l.T, b3[None, :], W3r.T)
  return out
